# baseline (speedup 1.0000x reference)
import functools

import jax
import jax.numpy as jnp
from jax import lax
from jax.experimental import pallas as pl
from jax.experimental.pallas import tpu as pltpu
from jax.experimental.pallas import tpu_sc as plsc

B = 4096
N_SPARSE = 26
HIST = 50
N_DENSE = 13
VOCAB = 100000
DIM = 64
OUT_W = N_SPARSE * DIM + DIM + N_DENSE
OUT_P = 1744
TROW = OUT_P // 8

_info = plsc.get_sparse_core_info()
NC, NS, L = _info.num_cores, _info.num_subcores, _info.num_lanes
NW = NC * NS
RPW = B // NW
C = 16
NCHUNK = RPW // C

SP_N = C * N_SPARSE
VL_N = C * HIST
SP_SLICE = 104
VL_SLICE = 80


def _sc_body(spi_hbm, vli_hbm, len_hbm, dn_hbm, bias_hbm, table_hbm, out_hbm,
             spidx, vlidx, lenb, denb, biasb, gsp, gvl, outb,
             semsp, semvl, semo):
    wid = lax.axis_index("s") * NC + lax.axis_index("c")
    wbase = wid * RPW

    pltpu.sync_copy(bias_hbm, biasb)
    pltpu.sync_copy(spi_hbm.at[pl.ds(wbase * N_SPARSE, RPW * N_SPARSE)], spidx)
    pltpu.sync_copy(vli_hbm.at[pl.ds(wbase * HIST, RPW * HIST)], vlidx)
    pltpu.sync_copy(len_hbm.at[pl.ds(wbase, RPW)], lenb)
    pltpu.sync_copy(dn_hbm.at[pl.ds(wbase * N_DENSE, RPW * N_DENSE)],
                    denb.at[pl.ds(0, RPW * N_DENSE)])
    biasv = biasb[...]
    lanes = lax.iota(jnp.int32, L)
    lanes_hi = jnp.right_shift(lanes, 3)
    lanes_lo = jnp.bitwise_and(lanes, 7)

    def fire_sp(k):
        o0 = k * SP_N
        return [pltpu.async_copy(
            table_hbm.at[spidx.at[pl.ds(o0 + j * SP_SLICE, SP_SLICE)]],
            gsp.at[pl.ds(j * SP_SLICE, SP_SLICE)], semsp)
            for j in range(SP_N // SP_SLICE)]

    def fire_vl(k):
        v0 = k * VL_N
        return [pltpu.async_copy(
            table_hbm.at[vlidx.at[pl.ds(v0 + j * VL_SLICE, VL_SLICE)]],
            gvl.at[pl.ds(j * VL_SLICE, VL_SLICE)], semvl)
            for j in range(VL_N // VL_SLICE)]

    pend_o = None
    sp_cp = fire_sp(0)
    vl_cp = fire_vl(0)
    for k in range(NCHUNK):
        brow = k * C

        for cp in sp_cp:
            cp.wait()
        if pend_o is not None:
            for cp in pend_o:
                cp.wait()

        def row_body(c, carry):
            colv = jnp.zeros((L,), jnp.int32) + c

            def f_body(f, carry2):
                row = c * N_SPARSE + f
                rb = f * DIM
                for d in range(DIM // L):
                    v = gsp[row, pl.ds(d * L, L)]
                    plsc.store_scatter(
                        outb, [((rb + d * L) >> 3) + lanes_hi, lanes_lo, colv],
                        v + biasv)
                return carry2
            lax.fori_loop(0, N_SPARSE, f_body, 0)
            return carry
        lax.fori_loop(0, C, row_body, 0)

        if k + 1 < NCHUNK:
            sp_next = fire_sp(k + 1)
        for cp in vl_cp:
            cp.wait()

        def pool_body(c, carry):
            colv = jnp.zeros((L,), jnp.int32) + c
            lenv = plsc.load_gather(lenb, [colv + brow])
            inv = 1.0 / jnp.maximum(lenv.astype(jnp.float32), 1.0)
            zero = jnp.zeros((L,), jnp.float32)
            vbase = c * HIST

            def j_body(j, acc):
                m = lenv > j
                r = vbase + j
                return tuple(
                    acc[d] + jnp.where(m, gvl[r, pl.ds(d * L, L)], zero)
                    for d in range(DIM // L))
            acc = lax.fori_loop(0, HIST, j_body, (zero,) * (DIM // L))
            pb = N_SPARSE * DIM
            for d in range(DIM // L):
                plsc.store_scatter(
                    outb, [((pb + d * L) >> 3) + lanes_hi, lanes_lo, colv],
                    acc[d] * inv + biasv)

            dmask = lanes < N_DENSE
            dv = plsc.load_gather(denb, [(brow + c) * N_DENSE + lanes],
                                  mask=dmask)
            plsc.store_scatter(
                outb, [((pb + DIM) >> 3) + lanes_hi, lanes_lo, colv],
                dv + biasv, mask=dmask)
            return carry
        lax.fori_loop(0, C, pool_body, 0)

        if k + 1 < NCHUNK:
            sp_cp = sp_next
            vl_cp = fire_vl(k + 1)
        pend_o = [pltpu.async_copy(
            outb, out_hbm.at[:, wid, :, pl.ds(brow, C)], semo)]
    for cp in pend_o:
        cp.wait()


@jax.jit
def _run(spi, vli, lens, dn, bias16, tflat):
    table = tflat.reshape(VOCAB, DIM)
    mesh = plsc.VectorSubcoreMesh(core_axis_name="c", subcore_axis_name="s")
    k = functools.partial(
        pl.kernel,
        out_type=jax.ShapeDtypeStruct((TROW, B // 128, 8, 128), jnp.float32),
        mesh=mesh,
        compiler_params=pltpu.CompilerParams(use_tc_tiling_on_sc=False,
                                             needs_layout_passes=False),
        scratch_types=[
            pltpu.VMEM((RPW * N_SPARSE,), jnp.int32),
            pltpu.VMEM((RPW * HIST,), jnp.int32),
            pltpu.VMEM((RPW,), jnp.int32),
            pltpu.VMEM((RPW * N_DENSE + L,), jnp.float32),
            pltpu.VMEM((L,), jnp.float32),
            pltpu.VMEM((SP_N, DIM), jnp.float32),
            pltpu.VMEM((VL_N, DIM), jnp.float32),
            pltpu.VMEM((TROW, 8, C), jnp.float32),
            pltpu.SemaphoreType.DMA,
            pltpu.SemaphoreType.DMA,
            pltpu.SemaphoreType.DMA,
        ],
    )(_sc_body)
    res = k(spi, vli, lens, dn, bias16, table)
    return res.transpose(1, 3, 0, 2).reshape(B, OUT_P)[:, :OUT_W]


def kernel(sparse_idx, varlen_idx, varlen_len, dense, emb_table, out_bias):
    spi = sparse_idx.astype(jnp.int32).reshape(-1)
    vli = varlen_idx.astype(jnp.int32).reshape(-1)
    lens = varlen_len.astype(jnp.int32)
    dn = dense.astype(jnp.float32).reshape(-1)
    bias16 = jnp.zeros((L,), jnp.float32) + out_bias.astype(jnp.float32)
    return _run(spi, vli, lens, dn, bias16,
                emb_table.astype(jnp.float32).reshape(-1))


# speedup vs baseline: 1.0465x; 1.0465x over previous
import functools

import jax
import jax.numpy as jnp
from jax import lax
from jax.experimental import pallas as pl
from jax.experimental.pallas import tpu as pltpu
from jax.experimental.pallas import tpu_sc as plsc

B = 4096
N_SPARSE = 26
HIST = 50
N_DENSE = 13
VOCAB = 100000
DIM = 64
OUT_W = N_SPARSE * DIM + DIM + N_DENSE
OUT_P = 1744
TROW = OUT_P // 8

_info = plsc.get_sparse_core_info()
NC, NS, L = _info.num_cores, _info.num_subcores, _info.num_lanes
NW = NC * NS
RPW = B // NW
C = 16
NCHUNK = RPW // C

SP_N = C * N_SPARSE
VL_N = C * HIST
SP_SLICES = (128, 128, 128, 32)
VL_SLICES = (128, 128, 128, 128, 128, 128, 32)


def _sc_body(spi_hbm, vli_hbm, len_hbm, dn_hbm, bias_hbm, table_hbm, out_hbm,
             spidx, vlidx, lenb, denb, biasb, gsp, gvl, outb,
             semsp, semvl, semo):
    wid = lax.axis_index("s") * NC + lax.axis_index("c")
    wbase = wid * RPW

    pltpu.sync_copy(bias_hbm, biasb)
    pltpu.sync_copy(spi_hbm.at[pl.ds(wbase * N_SPARSE, RPW * N_SPARSE)], spidx)
    pltpu.sync_copy(vli_hbm.at[pl.ds(wbase * HIST, RPW * HIST)], vlidx)
    pltpu.sync_copy(len_hbm.at[pl.ds(wbase, RPW)], lenb)
    pltpu.sync_copy(dn_hbm.at[pl.ds(wbase * N_DENSE, RPW * N_DENSE)],
                    denb.at[pl.ds(0, RPW * N_DENSE)])
    biasv = biasb[...]
    lanes = lax.iota(jnp.int32, L)
    lanes_hi = jnp.right_shift(lanes, 3)
    lanes_lo = jnp.bitwise_and(lanes, 7)

    def fire_sp(k):
        cps, o = [], 0
        for n in SP_SLICES:
            cps.append(pltpu.async_copy(
                table_hbm.at[spidx.at[pl.ds(k * SP_N + o, n)]],
                gsp.at[pl.ds(o, n)], semsp))
            o += n
        return cps

    def fire_vl(k):
        cps, o = [], 0
        for n in VL_SLICES:
            cps.append(pltpu.async_copy(
                table_hbm.at[vlidx.at[pl.ds(k * VL_N + o, n)]],
                gvl.at[pl.ds(o, n)], semvl))
            o += n
        return cps

    pend_o = None
    sp_cp = fire_sp(0)
    vl_cp = fire_vl(0)
    for k in range(NCHUNK):
        brow = k * C

        for cp in sp_cp:
            cp.wait()
        if pend_o is not None:
            for cp in pend_o:
                cp.wait()

        def row_body(c, carry):
            colv = jnp.zeros((L,), jnp.int32) + c

            def f_body(f2, carry2):
                for h in range(2):
                    f = f2 * 2 + h
                    row = c * N_SPARSE + f
                    rb = f * DIM
                    for d in range(DIM // L):
                        v = gsp[row, pl.ds(d * L, L)]
                        plsc.store_scatter(
                            outb,
                            [((rb + d * L) >> 3) + lanes_hi, lanes_lo, colv],
                            v + biasv)
                return carry2
            lax.fori_loop(0, N_SPARSE // 2, f_body, 0)
            return carry
        lax.fori_loop(0, C, row_body, 0)

        if k + 1 < NCHUNK:
            sp_next = fire_sp(k + 1)
        for cp in vl_cp:
            cp.wait()

        def pool_body(c, carry):
            colv = jnp.zeros((L,), jnp.int32) + c
            lenv = plsc.load_gather(lenb, [colv + brow])
            inv = 1.0 / jnp.maximum(lenv.astype(jnp.float32), 1.0)
            zero = jnp.zeros((L,), jnp.float32)
            vbase = c * HIST

            def j_body(j2, acc):
                for h in range(2):
                    j = j2 * 2 + h
                    m = lenv > j
                    r = vbase + j
                    acc = tuple(
                        acc[d] + jnp.where(m, gvl[r, pl.ds(d * L, L)], zero)
                        for d in range(DIM // L))
                return acc
            acc = lax.fori_loop(0, HIST // 2, j_body, (zero,) * (DIM // L))
            pb = N_SPARSE * DIM
            for d in range(DIM // L):
                plsc.store_scatter(
                    outb, [((pb + d * L) >> 3) + lanes_hi, lanes_lo, colv],
                    acc[d] * inv + biasv)

            dmask = lanes < N_DENSE
            dv = plsc.load_gather(denb, [(brow + c) * N_DENSE + lanes],
                                  mask=dmask)
            plsc.store_scatter(
                outb, [((pb + DIM) >> 3) + lanes_hi, lanes_lo, colv],
                dv + biasv, mask=dmask)
            return carry
        lax.fori_loop(0, C, pool_body, 0)

        if k + 1 < NCHUNK:
            sp_cp = sp_next
            vl_cp = fire_vl(k + 1)
        pend_o = [pltpu.async_copy(
            outb, out_hbm.at[:, wid, :, pl.ds(brow, C)], semo)]
    for cp in pend_o:
        cp.wait()


@jax.jit
def _run(spi, vli, lens, dn, bias16, tflat):
    table = tflat.reshape(VOCAB, DIM)
    mesh = plsc.VectorSubcoreMesh(core_axis_name="c", subcore_axis_name="s")
    k = functools.partial(
        pl.kernel,
        out_type=jax.ShapeDtypeStruct((TROW, B // 128, 8, 128), jnp.float32),
        mesh=mesh,
        compiler_params=pltpu.CompilerParams(use_tc_tiling_on_sc=False,
                                             needs_layout_passes=False),
        scratch_types=[
            pltpu.VMEM((RPW * N_SPARSE,), jnp.int32),
            pltpu.VMEM((RPW * HIST,), jnp.int32),
            pltpu.VMEM((RPW,), jnp.int32),
            pltpu.VMEM((RPW * N_DENSE + L,), jnp.float32),
            pltpu.VMEM((L,), jnp.float32),
            pltpu.VMEM((SP_N, DIM), jnp.float32),
            pltpu.VMEM((VL_N, DIM), jnp.float32),
            pltpu.VMEM((TROW, 8, C), jnp.float32),
            pltpu.SemaphoreType.DMA,
            pltpu.SemaphoreType.DMA,
            pltpu.SemaphoreType.DMA,
        ],
    )(_sc_body)
    res = k(spi, vli, lens, dn, bias16, table)
    return res.transpose(1, 3, 0, 2).reshape(B, OUT_P)[:, :OUT_W]


def kernel(sparse_idx, varlen_idx, varlen_len, dense, emb_table, out_bias):
    spi = sparse_idx.astype(jnp.int32).reshape(-1)
    vli = varlen_idx.astype(jnp.int32).reshape(-1)
    lens = varlen_len.astype(jnp.int32)
    dn = dense.astype(jnp.float32).reshape(-1)
    bias16 = jnp.zeros((L,), jnp.float32) + out_bias.astype(jnp.float32)
    return _run(spi, vli, lens, dn, bias16,
                emb_table.astype(jnp.float32).reshape(-1))


# speedup vs baseline: 1.0475x; 1.0010x over previous
import functools

import jax
import jax.numpy as jnp
from jax import lax
from jax.experimental import pallas as pl
from jax.experimental.pallas import tpu as pltpu
from jax.experimental.pallas import tpu_sc as plsc

B = 4096
N_SPARSE = 26
HIST = 50
N_DENSE = 13
VOCAB = 100000
DIM = 64
OUT_W = N_SPARSE * DIM + DIM + N_DENSE
OUT_P = 1744
TROW = OUT_P // 8

_info = plsc.get_sparse_core_info()
NC, NS, L = _info.num_cores, _info.num_subcores, _info.num_lanes
NW = NC * NS
RPW = B // NW
C = 16
NCHUNK = RPW // C

SP_N = C * N_SPARSE
VL_N = C * HIST
SP_SLICES = (128, 128, 128, 32)
VL_SLICES = (128, 128, 128, 128, 128, 128, 32)


def _sc_body(spi_hbm, vli_hbm, len_hbm, dn_hbm, bias_hbm, table_hbm, out_hbm,
             spidx, vlidx, lenb, denb, biasb, gsp, gvl, outb,
             semsp, semvl, semo):
    wid = lax.axis_index("s") * NC + lax.axis_index("c")
    wbase = wid * RPW

    pltpu.sync_copy(bias_hbm, biasb)
    pltpu.sync_copy(spi_hbm.at[pl.ds(wbase * N_SPARSE, RPW * N_SPARSE)], spidx)
    pltpu.sync_copy(vli_hbm.at[pl.ds(wbase * HIST, RPW * HIST)], vlidx)
    pltpu.sync_copy(len_hbm.at[pl.ds(wbase, RPW)], lenb)
    pltpu.sync_copy(dn_hbm.at[pl.ds(wbase * N_DENSE, RPW * N_DENSE)],
                    denb.at[pl.ds(0, RPW * N_DENSE)])
    biasv = biasb[...]
    lanes = lax.iota(jnp.int32, L)
    lanes_hi = jnp.right_shift(lanes, 3)
    lanes_lo = jnp.bitwise_and(lanes, 7)

    def fire_sp(k):
        cps, o = [], 0
        for n in SP_SLICES:
            cps.append(pltpu.async_copy(
                table_hbm.at[spidx.at[pl.ds(k * SP_N + o, n)]],
                gsp.at[pl.ds(o, n)], semsp))
            o += n
        return cps

    def fire_vl(k):
        cps, o = [], 0
        for n in VL_SLICES:
            cps.append(pltpu.async_copy(
                table_hbm.at[vlidx.at[pl.ds(k * VL_N + o, n)]],
                gvl.at[pl.ds(o, n)], semvl))
            o += n
        return cps

    pend_o = None
    sp_cp = fire_sp(0)
    vl_cp = fire_vl(0)
    for k in range(NCHUNK):
        brow = k * C

        for cp in sp_cp:
            cp.wait()
        if pend_o is not None:
            for cp in pend_o:
                cp.wait()

        def row_body(c, carry):
            colv = jnp.zeros((L,), jnp.int32) + c

            def f_body(f2, carry2):
                for h in range(2):
                    f = f2 * 2 + h
                    row = c * N_SPARSE + f
                    rb = f * DIM
                    for d in range(DIM // L):
                        v = gsp[row, pl.ds(d * L, L)]
                        plsc.store_scatter(
                            outb,
                            [((rb + d * L) >> 3) + lanes_hi, lanes_lo, colv],
                            v + biasv)
                return carry2
            lax.fori_loop(0, N_SPARSE // 2, f_body, 0)
            return carry
        lax.fori_loop(0, C, row_body, 0)

        if k + 1 < NCHUNK:
            sp_next = fire_sp(k + 1)
        for cp in vl_cp:
            cp.wait()

        def pool_body(c, carry):
            colv = jnp.zeros((L,), jnp.int32) + c
            lenv = plsc.load_gather(lenb, [colv + brow])
            inv = 1.0 / jnp.maximum(lenv.astype(jnp.float32), 1.0)
            zero = jnp.zeros((L,), jnp.float32)
            vbase = c * HIST

            def j_body(j2, acc):
                for h in range(5):
                    j = j2 * 5 + h
                    m = lenv > j
                    r = vbase + j
                    acc = tuple(
                        acc[d] + jnp.where(m, gvl[r, pl.ds(d * L, L)], zero)
                        for d in range(DIM // L))
                return acc
            acc = lax.fori_loop(0, HIST // 5, j_body, (zero,) * (DIM // L))
            pb = N_SPARSE * DIM
            for d in range(DIM // L):
                plsc.store_scatter(
                    outb, [((pb + d * L) >> 3) + lanes_hi, lanes_lo, colv],
                    acc[d] * inv + biasv)

            dmask = lanes < N_DENSE
            dv = plsc.load_gather(denb, [(brow + c) * N_DENSE + lanes],
                                  mask=dmask)
            plsc.store_scatter(
                outb, [((pb + DIM) >> 3) + lanes_hi, lanes_lo, colv],
                dv + biasv, mask=dmask)
            return carry
        lax.fori_loop(0, C, pool_body, 0)

        if k + 1 < NCHUNK:
            sp_cp = sp_next
            vl_cp = fire_vl(k + 1)
        pend_o = [pltpu.async_copy(
            outb, out_hbm.at[:, wid, :, pl.ds(brow, C)], semo)]
    for cp in pend_o:
        cp.wait()


@jax.jit
def _run(spi, vli, lens, dn, bias16, tflat):
    table = tflat.reshape(VOCAB, DIM)
    mesh = plsc.VectorSubcoreMesh(core_axis_name="c", subcore_axis_name="s")
    k = functools.partial(
        pl.kernel,
        out_type=jax.ShapeDtypeStruct((TROW, B // 128, 8, 128), jnp.float32),
        mesh=mesh,
        compiler_params=pltpu.CompilerParams(use_tc_tiling_on_sc=False,
                                             needs_layout_passes=False),
        scratch_types=[
            pltpu.VMEM((RPW * N_SPARSE,), jnp.int32),
            pltpu.VMEM((RPW * HIST,), jnp.int32),
            pltpu.VMEM((RPW,), jnp.int32),
            pltpu.VMEM((RPW * N_DENSE + L,), jnp.float32),
            pltpu.VMEM((L,), jnp.float32),
            pltpu.VMEM((SP_N, DIM), jnp.float32),
            pltpu.VMEM((VL_N, DIM), jnp.float32),
            pltpu.VMEM((TROW, 8, C), jnp.float32),
            pltpu.SemaphoreType.DMA,
            pltpu.SemaphoreType.DMA,
            pltpu.SemaphoreType.DMA,
        ],
    )(_sc_body)
    res = k(spi, vli, lens, dn, bias16, table)
    return res.transpose(1, 3, 0, 2).reshape(B, OUT_P)[:, :OUT_W]


def kernel(sparse_idx, varlen_idx, varlen_len, dense, emb_table, out_bias):
    spi = sparse_idx.astype(jnp.int32).reshape(-1)
    vli = varlen_idx.astype(jnp.int32).reshape(-1)
    lens = varlen_len.astype(jnp.int32)
    dn = dense.astype(jnp.float32).reshape(-1)
    bias16 = jnp.zeros((L,), jnp.float32) + out_bias.astype(jnp.float32)
    return _run(spi, vli, lens, dn, bias16,
                emb_table.astype(jnp.float32).reshape(-1))
